# baseline (speedup 1.0000x reference)
import functools

import jax
import jax.numpy as jnp
from jax import lax
from jax.experimental import pallas as pl
from jax.experimental.pallas import tpu as pltpu


def _fc_body(h_ref, w1_ref, b1_ref, w2_ref, b2_ref, w3_ref, b3_ref, o_ref):
    h = h_ref[...]
    h = jax.nn.relu(jnp.dot(h, w1_ref[...], preferred_element_type=jnp.float32) + b1_ref[...])
    h = jax.nn.relu(jnp.dot(h, w2_ref[...], preferred_element_type=jnp.float32) + b2_ref[...])
    o_ref[...] = jnp.dot(h, w3_ref[...], preferred_element_type=jnp.float32) + b3_ref[...]


def _fc_head(h, fcW1, fcb1, fcW2, fcb2, fcW3, fcb3):
    B = h.shape[0]
    return pl.pallas_call(
        _fc_body,
        out_shape=jax.ShapeDtypeStruct((B, fcW3.shape[1]), jnp.float32),
    )(h, fcW1, fcb1[None, :], fcW2, fcb2[None, :], fcW3, fcb3[None, :])


def _lap_weights(ei, ew, n):
    src, dst = ei[0], ei[1]
    deg = jnp.zeros((n,), dtype=ew.dtype).at[dst].add(ew)
    dis = jnp.where(deg > 0, lax.rsqrt(jnp.maximum(deg, 1e-12)), 0.0)
    return -ew * dis[src] * dis[dst]


def _lap_apply(h, src, dst, w):
    msg = h[:, src, :] * w[None, :, None]
    return jnp.zeros_like(h).at[:, dst, :].add(msg)


def _cheb(h, ei, ew, W, b):
    n = h.shape[1]
    src, dst = ei[0], ei[1]
    w = _lap_weights(ei, ew, n)
    Tx0 = h
    out = jnp.einsum('bnc,cd->bnd', Tx0, W[0])
    Tx1 = _lap_apply(h, src, dst, w)
    out = out + jnp.einsum('bnc,cd->bnd', Tx1, W[1])
    for k in range(2, W.shape[0]):
        Tx2 = 2.0 * _lap_apply(Tx1, src, dst, w) - Tx0
        out = out + jnp.einsum('bnc,cd->bnd', Tx2, W[k])
        Tx0, Tx1 = Tx1, Tx2
    return out + b[None, None, :]


def _bn_relu(h, g, be):
    Bn, N, C = h.shape
    hf = h.reshape(Bn * N, C)
    mean = hf.mean(axis=0)
    var = hf.var(axis=0)
    y = (hf - mean) / jnp.sqrt(var + 1e-5) * g + be
    return jax.nn.relu(y).reshape(Bn, N, C)


def _pool(h):
    Bn, N, C = h.shape
    return h.reshape(Bn, N // 4, 4, C).mean(axis=2)


_CONV_NSIDES = [64, 64, 64, 32, 32, 16, 16]
_POOL_AFTER = [False, False, True, False, True, False, True]


def kernel(x, ei64, ew64, ei32, ew32, ei16, ew16, W1, b1, g1, be1, W2, b2, g2, be2, W3, b3, g3, be3, W4, b4, g4, be4, W5, b5, g5, be5, W6, b6, g6, be6, W7, b7, g7, be7, fcW1, fcb1, fcW2, fcb2, fcW3, fcb3):
    graphs = {64: (ei64, ew64), 32: (ei32, ew32), 16: (ei16, ew16)}
    Ws = [W1, W2, W3, W4, W5, W6, W7]
    bs = [b1, b2, b3, b4, b5, b6, b7]
    gs = [g1, g2, g3, g4, g5, g6, g7]
    bes = [be1, be2, be3, be4, be5, be6, be7]
    h = jnp.transpose(x, (0, 2, 1))
    for l in range(7):
        ei, ew = graphs[_CONV_NSIDES[l]]
        h = _cheb(h, ei, ew, Ws[l], bs[l])
        h = _bn_relu(h, gs[l], bes[l])
        if _POOL_AFTER[l]:
            h = _pool(h)
    h = h.mean(axis=1)
    return _fc_head(h, fcW1, fcb1, fcW2, fcb2, fcW3, fcb3)


# speedup vs baseline: 4.8436x; 4.8436x over previous
import functools

import jax
import jax.numpy as jnp
from jax import lax
from jax.experimental import pallas as pl
from jax.experimental.pallas import tpu as pltpu
from jax.experimental.pallas import tpu_sc as plsc


_NW = 32


def _block_params(n):
    if n == 49152:
        return 64, 640
    if n == 12288:
        return 32, 352
    return 8, 112


def _prep_graph(ei, ew, n):
    RB, ECAP = _block_params(n)
    src, dst = ei[0], ei[1]
    e = src.shape[0]
    perm = jnp.argsort(dst)
    srcS, dstS, ewS = src[perm], dst[perm], ew[perm]
    starts = jnp.searchsorted(dstS, jnp.arange(n + 1, dtype=jnp.int32)).astype(jnp.int32)
    DCAP = 40
    rdeg = starts[1:] - starts[:-1]
    j = jnp.arange(DCAP, dtype=jnp.int32)
    rg = jnp.clip(starts[:-1, None] + j[None, :], 0, e - 1)
    deg = jnp.where(j[None, :] < rdeg[:, None], ewS[rg], 0.0).sum(axis=1)
    dis = jnp.where(deg > 0, lax.rsqrt(jnp.maximum(deg, 1e-12)), 0.0)
    wS = -ewS * dis[srcS] * dis[dstS]
    nblk = n // RB
    b0 = starts[jnp.arange(nblk) * RB]
    b1 = starts[jnp.arange(nblk) * RB + RB]
    cnt = b1 - b0
    chb = (cnt + 15) // 16
    q = jnp.arange(ECAP, dtype=jnp.int32)
    lane, cidx = q % 16, q // 16
    p = cidx[None, :] + lane[None, :] * chb[:, None]
    valid = (p < cnt[:, None]) & (cidx[None, :] < chb[:, None])
    gidx = jnp.clip(b0[:, None] + p, 0, e - 1)
    srcP = jnp.where(valid, srcS[gidx], 0).astype(jnp.int32)
    wP = jnp.where(valid, wS[gidx], 0.0).astype(jnp.float32)
    rloc = dstS[gidx] - (jnp.arange(nblk, dtype=jnp.int32) * RB)[:, None]
    dstP = jnp.where(valid, rloc, 0).astype(jnp.int32)
    return srcP, wP, dstP


@functools.lru_cache(maxsize=None)
def _make_sc_apply(n, f, affine):
    RB, ECAP = _block_params(n)
    nblk = n // RB
    kb = nblk // _NW
    nch = ECAP // 16
    SR = min(RB, 16)
    mesh = plsc.VectorSubcoreMesh(core_axis_name="c", subcore_axis_name="s")

    @functools.partial(
        pl.kernel, mesh=mesh,
        compiler_params=pltpu.CompilerParams(needs_layout_passes=False),
        out_type=jax.ShapeDtypeStruct((n * f,), jnp.float32),
        scratch_types=[
            pltpu.VMEM((RB * f,), jnp.float32),
            pltpu.VMEM((16, f), jnp.float32),
            pltpu.VMEM((ECAP,), jnp.int32),
            pltpu.VMEM((ECAP,), jnp.float32),
            pltpu.VMEM((ECAP,), jnp.int32),
            pltpu.SemaphoreType.DMA,
        ],
    )
    def sc_apply(h_hbm, srcP_hbm, wP_hbm, dstP_hbm, aux_hbm, out_hbm,
                 acc, rowbuf, srcloc, wloc, dstloc, sem):
        wid = lax.axis_index("s") * 2 + lax.axis_index("c")
        lane16 = jnp.arange(16, dtype=jnp.int32)
        zro = jnp.zeros((16,), jnp.float32)
        nfc = f // 16

        def block_body(i, carry):
            blk = wid * kb + i
            pltpu.sync_copy(srcP_hbm.at[blk], srcloc)
            pltpu.sync_copy(wP_hbm.at[blk], wloc)
            pltpu.sync_copy(dstP_hbm.at[blk], dstloc)

            def zcol(j, c3):
                acc[pl.ds(pl.multiple_of(j * 16, 16), 16)] = zro
                return c3
            lax.fori_loop(0, RB * nfc, zcol, carry)

            def chunk_body(c, c2):
                off = pl.multiple_of(c * 16, 16)
                idxv = srcloc[pl.ds(off, 16)]
                pltpu.async_copy(h_hbm.at[idxv], rowbuf, sem).wait()
                wsp, rbase = [], []
                for l in range(16):
                    el = jnp.full((16,), off + l, jnp.int32)
                    wsp.append(plsc.load_gather(wloc, [el]))
                    rbase.append(plsc.load_gather(dstloc, [el]) * f + lane16)

                def fbody(fc, c3):
                    f0 = pl.multiple_of(fc * 16, 16)
                    fv = jnp.full((16,), f0, jnp.int32)
                    for l in range(16):
                        vec = rowbuf[l, pl.ds(f0, 16)]
                        plsc.addupdate_scatter(acc, [rbase[l] + fv], vec * wsp[l])
                    return c3
                return lax.fori_loop(0, nfc, fbody, c2)
            lax.fori_loop(0, nch, chunk_body, carry)

            base = pl.multiple_of(blk * (RB * f), 16)
            if affine:
                for sub in range(RB // SR):
                    pltpu.sync_copy(aux_hbm.at[pl.ds(blk * RB + sub * SR, SR)],
                                    rowbuf.at[pl.ds(0, SR)])
                    for rr in range(SR):
                        ab = pl.multiple_of((sub * SR + rr) * f, 16)

                        def acol(fc, c3, _ab=ab, _rr=rr):
                            f0 = pl.multiple_of(fc * 16, 16)
                            s = pl.ds(_ab + f0, 16)
                            acc[s] = 2.0 * acc[s] - rowbuf[_rr, pl.ds(f0, 16)]
                            return c3
                        lax.fori_loop(0, nfc, acol, carry)
            pltpu.sync_copy(acc, out_hbm.at[pl.ds(base, RB * f)])
            return carry

        lax.fori_loop(0, kb, block_body, 0)

    return sc_apply


def _cheb_layer(h2d, srcP, wP, dstP, W, b, n, B):
    cin, cout = W.shape[1], W.shape[2]
    f = B * cin
    fp = max(f, 128)
    cp = fp // B
    if fp != f:
        h2d = jnp.pad(h2d.reshape(n, B, cin),
                      ((0, 0), (0, 0), (0, cp - cin))).reshape(n, fp)
    apply_plain = _make_sc_apply(n, fp, False)
    apply_affine = _make_sc_apply(n, fp, True)

    def mm(t2d, Wk):
        return jnp.dot(t2d.reshape(n * B, cp)[:, :cin], Wk,
                       preferred_element_type=jnp.float32,
                       precision=lax.Precision.HIGHEST)

    Tx0 = h2d
    out = mm(Tx0, W[0])
    Tx1 = apply_plain(h2d, srcP, wP, dstP, h2d).reshape(n, fp)
    out = out + mm(Tx1, W[1])
    for k in range(2, W.shape[0]):
        Tx2 = apply_affine(Tx1, srcP, wP, dstP, Tx0).reshape(n, fp)
        out = out + mm(Tx2, W[k])
        Tx0, Tx1 = Tx1, Tx2
    return (out + b[None, :]).reshape(n, B, cout)


def _bn_relu_2d(h3, g, be):
    n, B, C = h3.shape
    hf = h3.reshape(n * B, C)
    mean = hf.mean(axis=0)
    var = hf.var(axis=0)
    y = (hf - mean) / jnp.sqrt(var + 1e-5) * g + be
    return jax.nn.relu(y).reshape(n, B, C)


def _fc_body(h_ref, w1_ref, b1_ref, w2_ref, b2_ref, w3_ref, b3_ref, o_ref):
    h = h_ref[...]
    h = jax.nn.relu(jnp.dot(h, w1_ref[...], preferred_element_type=jnp.float32) + b1_ref[...])
    h = jax.nn.relu(jnp.dot(h, w2_ref[...], preferred_element_type=jnp.float32) + b2_ref[...])
    o_ref[...] = jnp.dot(h, w3_ref[...], preferred_element_type=jnp.float32) + b3_ref[...]


def _fc_head(h, fcW1, fcb1, fcW2, fcb2, fcW3, fcb3):
    B = h.shape[0]
    return pl.pallas_call(
        _fc_body,
        out_shape=jax.ShapeDtypeStruct((B, fcW3.shape[1]), jnp.float32),
    )(h, fcW1, fcb1[None, :], fcW2, fcb2[None, :], fcW3, fcb3[None, :])


_CONV_NSIDES = [64, 64, 64, 32, 32, 16, 16]
_POOL_AFTER = [False, False, True, False, True, False, True]


def kernel(x, ei64, ew64, ei32, ew32, ei16, ew16, W1, b1, g1, be1, W2, b2, g2, be2, W3, b3, g3, be3, W4, b4, g4, be4, W5, b5, g5, be5, W6, b6, g6, be6, W7, b7, g7, be7, fcW1, fcb1, fcW2, fcb2, fcW3, fcb3):
    B = x.shape[0]
    packed = {}
    for ns, ei, ew in ((64, ei64, ew64), (32, ei32, ew32), (16, ei16, ew16)):
        packed[ns] = _prep_graph(ei, ew, 12 * ns * ns)
    Ws = [W1, W2, W3, W4, W5, W6, W7]
    bs = [b1, b2, b3, b4, b5, b6, b7]
    gs = [g1, g2, g3, g4, g5, g6, g7]
    bes = [be1, be2, be3, be4, be5, be6, be7]

    h2d = jnp.transpose(x[:, 0, :], (1, 0))
    for l in range(7):
        ns = _CONV_NSIDES[l]
        n = 12 * ns * ns
        srcP, wP, dstP = packed[ns]
        h3 = _cheb_layer(h2d, srcP, wP, dstP, Ws[l], bs[l], n, B)
        h3 = _bn_relu_2d(h3, gs[l], bes[l])
        if _POOL_AFTER[l]:
            n2, C = n // 4, h3.shape[2]
            h3 = h3.reshape(n2, 4, B, C).mean(axis=1)
            n = n2
        h2d = h3.reshape(n, -1)
    hB = h3.mean(axis=0)
    return _fc_head(hB, fcW1, fcb1, fcW2, fcb2, fcW3, fcb3)
